# Initial kernel scaffold; baseline (speedup 1.0000x reference)
import jax
import jax.numpy as jnp
from jax.experimental import pallas as pl


def kernel(coordinates, features, active_vertices):
    raise NotImplementedError("write your pallas kernel here")


# speedup vs baseline: 6.1709x; 6.1709x over previous
import functools

import jax
import jax.numpy as jnp
from jax import lax
from jax.experimental import pallas as pl
from jax.experimental.pallas import tpu as pltpu
from jax.experimental.pallas import tpu_sc as plsc
import numpy as np

B, V, S, F, K = 4, 1024, 16, 128, 16
MAXD = float(np.finfo(np.float32).max)
R = 256


def _topk_body(act_ref, rows_ref, cols_ref, vals_ref, idx_ref):
    b = pl.program_id(0)
    rblk = pl.program_id(1)
    act = act_ref[0, 0, 0]
    cr = rows_ref[0]
    cc = cols_ref[0]

    a = [jnp.abs(cr[:, s : s + 1] - cc[s : s + 1, :]) for s in range(S)]

    def _tree8(h):
        return ((h[0] + h[4]) + (h[2] + h[6])) + ((h[1] + h[5]) + (h[3] + h[7]))

    dist = _tree8(a[0:8]) + _tree8(a[8:16])

    col = lax.broadcasted_iota(jnp.int32, (R, V), 1)
    row = rblk * R + lax.broadcasted_iota(jnp.int32, (R, V), 0)
    valid = (row < act) & (col < act)
    dist = jnp.where(valid, dist, MAXD)

    vals_cols = []
    idx_cols = []
    inf = jnp.float32(jnp.inf)
    for _ in range(K):
        mv = jnp.min(dist, axis=1, keepdims=True)
        mi = jnp.min(jnp.where(dist == mv, col, V), axis=1, keepdims=True)
        vals_cols.append(mv)
        idx_cols.append(mi)
        dist = jnp.where(col == mi, inf, dist)
    vals_ref[0] = jnp.concatenate(vals_cols, axis=1)
    idx_ref[0] = jnp.concatenate(idx_cols, axis=1) + b * V


def _topk_call(coords, coords_t, active):
    return pl.pallas_call(
        _topk_body,
        grid=(B, V // R),
        in_specs=[
            pl.BlockSpec((1, 1, 1), lambda b, r: (b, 0, 0), memory_space=pltpu.SMEM),
            pl.BlockSpec((1, R, S), lambda b, r: (b, r, 0)),
            pl.BlockSpec((1, S, V), lambda b, r: (b, 0, 0)),
        ],
        out_specs=[
            pl.BlockSpec((1, R, K), lambda b, r: (b, r, 0)),
            pl.BlockSpec((1, R, K), lambda b, r: (b, r, 0)),
        ],
        out_shape=[
            jax.ShapeDtypeStruct((B, V, K), jnp.float32),
            jax.ShapeDtypeStruct((B, V, K), jnp.int32),
        ],
        compiler_params=pltpu.CompilerParams(
            dimension_semantics=("parallel", "parallel"),
        ),
    )(active.reshape(B, 1, 1), coords, coords_t)


NIDX = B * V * (K - 1)
NC, NS = 2, 16
NW = NC * NS
BPW = NIDX // NW
CH = 384
NCHUNK = BPW // CH


def _gather_call(table, idx_flat):
    mesh = plsc.VectorSubcoreMesh(core_axis_name="c", subcore_axis_name="s")

    @functools.partial(
        pl.kernel,
        mesh=mesh,
        out_type=jax.ShapeDtypeStruct((NIDX, F), jnp.float32),
        scratch_types=[
            pltpu.VMEM((BPW,), jnp.int32),
            pltpu.VMEM((CH, F), jnp.float32),
            pltpu.VMEM((CH, F), jnp.float32),
            pltpu.SemaphoreType.DMA,
            pltpu.SemaphoreType.DMA,
        ],
    )
    def gk(table_hbm, idx_hbm, out_hbm, idx_v, buf0, buf1, sem0, sem1):
        wid = lax.axis_index("s") * NC + lax.axis_index("c")
        base = wid * BPW
        pltpu.sync_copy(idx_hbm.at[pl.ds(base, BPW)], idx_v)
        bufs = (buf0, buf1)
        sems = (sem0, sem1)
        copies = [None, None]
        for c in range(NCHUNK + 1):
            if c < NCHUNK:
                cp = pltpu.async_copy(
                    table_hbm.at[idx_v.at[pl.ds(c * CH, CH)]], bufs[c % 2], sems[c % 2]
                )
                copies[c % 2] = cp
            if c > 0:
                copies[(c - 1) % 2].wait()
                pltpu.sync_copy(
                    bufs[(c - 1) % 2], out_hbm.at[pl.ds(base + (c - 1) * CH, CH)]
                )

    return gk(table, idx_flat)


def kernel(coordinates, features, active_vertices):
    coords_t = jnp.transpose(coordinates, (0, 2, 1))
    vals, idx = _topk_call(coordinates, coords_t, active_vertices)
    neighbour_distances = vals[:, :, 1:]
    idx_flat = idx[:, :, 1:].reshape(NIDX)
    table = features.reshape(B * V, F)
    nf = _gather_call(table, idx_flat)
    neighbour_features = nf.reshape(B, V, K - 1, F)
    return (neighbour_distances, neighbour_features)


# speedup vs baseline: 6.2919x; 1.0196x over previous
import functools

import jax
import jax.numpy as jnp
from jax import lax
from jax.experimental import pallas as pl
from jax.experimental.pallas import tpu as pltpu
from jax.experimental.pallas import tpu_sc as plsc
import numpy as np

B, V, S, F, K = 4, 1024, 16, 128, 16
MAXD = float(np.finfo(np.float32).max)
R = 256


def _topk_body(act_ref, rows_ref, cols_ref, vals_ref, idx_ref):
    b = pl.program_id(0)
    rblk = pl.program_id(1)
    act = act_ref[0, 0, 0]
    cr = rows_ref[0]
    cc = cols_ref[0]

    a = [jnp.abs(cr[:, s : s + 1] - cc[s : s + 1, :]) for s in range(S)]

    def _tree8(h):
        return ((h[0] + h[4]) + (h[2] + h[6])) + ((h[1] + h[5]) + (h[3] + h[7]))

    dist = _tree8(a[0:8]) + _tree8(a[8:16])

    col = lax.broadcasted_iota(jnp.int32, (R, V), 1)
    row = rblk * R + lax.broadcasted_iota(jnp.int32, (R, V), 0)
    inf = jnp.float32(jnp.inf)
    dist = jnp.where((row < act) & (col < act), dist, MAXD)
    dist = jnp.where(row == col, inf, dist)

    vals_cols = []
    idx_cols = []
    for _ in range(K - 1):
        mv = jnp.min(dist, axis=1, keepdims=True)
        mi = jnp.min(jnp.where(dist == mv, col, V), axis=1, keepdims=True)
        vals_cols.append(mv)
        idx_cols.append(mi)
        dist = jnp.where(col == mi, inf, dist)
    vals15 = jnp.concatenate(vals_cols, axis=1)
    idx15 = jnp.concatenate(idx_cols, axis=1)
    row15 = rblk * R + lax.broadcasted_iota(jnp.int32, (R, K - 1), 0)
    j15 = lax.broadcasted_iota(jnp.int32, (R, K - 1), 1)
    inactive = row15 >= act
    vals_ref[0] = jnp.where(inactive, MAXD, vals15)
    idx_ref[0] = jnp.where(inactive, j15 + 1, idx15) + b * V


def _topk_call(coords, coords_t, active):
    return pl.pallas_call(
        _topk_body,
        grid=(B, V // R),
        in_specs=[
            pl.BlockSpec((1, 1, 1), lambda b, r: (b, 0, 0), memory_space=pltpu.SMEM),
            pl.BlockSpec((1, R, S), lambda b, r: (b, r, 0)),
            pl.BlockSpec((1, S, V), lambda b, r: (b, 0, 0)),
        ],
        out_specs=[
            pl.BlockSpec((1, R, K - 1), lambda b, r: (b, r, 0)),
            pl.BlockSpec((1, R, K - 1), lambda b, r: (b, r, 0)),
        ],
        out_shape=[
            jax.ShapeDtypeStruct((B, V, K - 1), jnp.float32),
            jax.ShapeDtypeStruct((B, V, K - 1), jnp.int32),
        ],
        compiler_params=pltpu.CompilerParams(
            dimension_semantics=("parallel", "parallel"),
        ),
    )(active.reshape(B, 1, 1), coords, coords_t)


NIDX = B * V * (K - 1)
NC, NS = 2, 16
NW = NC * NS
BPW = NIDX // NW
CH = 480
NCHUNK = BPW // CH


def _gather_call(table, idx_flat):
    mesh = plsc.VectorSubcoreMesh(core_axis_name="c", subcore_axis_name="s")

    @functools.partial(
        pl.kernel,
        mesh=mesh,
        out_type=jax.ShapeDtypeStruct((NIDX, F), jnp.float32),
        scratch_types=[
            pltpu.VMEM((BPW,), jnp.int32),
            pltpu.VMEM((CH, F), jnp.float32),
            pltpu.VMEM((CH, F), jnp.float32),
            pltpu.SemaphoreType.DMA,
            pltpu.SemaphoreType.DMA,
        ],
    )
    def gk(table_hbm, idx_hbm, out_hbm, idx_v, buf0, buf1, sem0, sem1):
        wid = lax.axis_index("s") * NC + lax.axis_index("c")
        base = wid * BPW
        pltpu.sync_copy(idx_hbm.at[pl.ds(base, BPW)], idx_v)
        bufs = (buf0, buf1)
        sems = (sem0, sem1)
        copies = [None, None]
        for c in range(NCHUNK + 1):
            if c < NCHUNK:
                cp = pltpu.async_copy(
                    table_hbm.at[idx_v.at[pl.ds(c * CH, CH)]], bufs[c % 2], sems[c % 2]
                )
                copies[c % 2] = cp
            if c > 0:
                copies[(c - 1) % 2].wait()
                pltpu.sync_copy(
                    bufs[(c - 1) % 2], out_hbm.at[pl.ds(base + (c - 1) * CH, CH)]
                )

    return gk(table, idx_flat)


def kernel(coordinates, features, active_vertices):
    coords_t = jnp.transpose(coordinates, (0, 2, 1))
    neighbour_distances, idx = _topk_call(coordinates, coords_t, active_vertices)
    idx_flat = idx.reshape(NIDX)
    table = features.reshape(B * V, F)
    nf = _gather_call(table, idx_flat)
    neighbour_features = nf.reshape(B, V, K - 1, F)
    return (neighbour_distances, neighbour_features)
